# initial kernel scaffold (unmeasured)
import jax
import jax.numpy as jnp
from jax import lax
from jax.experimental import pallas as pl
from jax.experimental.pallas import tpu as pltpu

N_DEV = 8
EPS = 1e-5
N_GLOBAL = 4096


def kernel(x, gamma):
    m, n = x.shape
    gamma_row = gamma.reshape(1, n)

    def body(x_ref, g_ref, out_ref, comm_ref, send_sems, recv_sems):
        my_pos = lax.axis_index("i")
        right = lax.rem(my_pos + 1, N_DEV)

        xv = x_ref[:, :]
        partial = jnp.sum(xv * xv, axis=1, keepdims=True)
        comm_ref[pl.ds(my_pos, 1), :, :] = partial[None]

        for h in range(N_DEV - 1):
            src_idx = lax.rem(my_pos - h + N_DEV, N_DEV)
            rdma = pltpu.make_async_remote_copy(
                src_ref=comm_ref.at[src_idx],
                dst_ref=comm_ref.at[src_idx],
                send_sem=send_sems.at[h],
                recv_sem=recv_sems.at[h],
                device_id=(right,),
                device_id_type=pl.DeviceIdType.MESH,
            )
            rdma.start()
            rdma.wait()

        total = jnp.sum(comm_ref[:, :, :], axis=0)
        inv = lax.rsqrt(total / N_GLOBAL + EPS)
        out_ref[:, :] = xv * g_ref[0, :][None, :] * inv

    return pl.pallas_call(
        body,
        out_shape=jax.ShapeDtypeStruct((m, n), jnp.float32),
        in_specs=[
            pl.BlockSpec(memory_space=pltpu.VMEM),
            pl.BlockSpec(memory_space=pltpu.VMEM),
        ],
        out_specs=pl.BlockSpec(memory_space=pltpu.VMEM),
        scratch_shapes=[
            pltpu.VMEM((N_DEV, m, 1), jnp.float32),
            pltpu.SemaphoreType.DMA((N_DEV - 1,)),
            pltpu.SemaphoreType.DMA((N_DEV - 1,)),
        ],
        compiler_params=pltpu.CompilerParams(collective_id=0),
    )(x, gamma_row)


# baseline (device time: 52610 ns/iter reference)
import jax
import jax.numpy as jnp
from jax import lax
from jax.experimental import pallas as pl
from jax.experimental.pallas import tpu as pltpu

N_DEV = 8
EPS = 1e-5
N_GLOBAL = 4096


def kernel(x, gamma):
    m, n = x.shape
    gamma_row = gamma.reshape(1, n)

    def body(x_ref, g_ref, out_ref, comm_ref, send_sems, recv_sems):
        my_pos = lax.axis_index("i")
        right = lax.rem(my_pos + 1, N_DEV)

        xv = x_ref[:, :]
        partial = jnp.sum(xv * xv, axis=1, keepdims=True)
        comm_ref[pl.ds(my_pos, 1), :, :] = partial[None]

        for h in range(N_DEV - 1):
            src_idx = lax.rem(my_pos - h + N_DEV, N_DEV)
            rdma = pltpu.make_async_remote_copy(
                src_ref=comm_ref.at[src_idx],
                dst_ref=comm_ref.at[src_idx],
                send_sem=send_sems.at[h],
                recv_sem=recv_sems.at[h],
                device_id=(right,),
                device_id_type=pl.DeviceIdType.MESH,
            )
            rdma.start()
            rdma.wait()

        total = jnp.sum(comm_ref[:, :, :], axis=0)
        inv = lax.rsqrt(total / N_GLOBAL + EPS)
        out_ref[:, :] = xv * g_ref[0, :][None, :] * inv

    return pl.pallas_call(
        body,
        out_shape=jax.ShapeDtypeStruct((m, n), jnp.float32),
        in_specs=[
            pl.BlockSpec(memory_space=pltpu.VMEM),
            pl.BlockSpec(memory_space=pltpu.VMEM),
        ],
        out_specs=pl.BlockSpec(memory_space=pltpu.VMEM),
        scratch_shapes=[
            pltpu.VMEM((N_DEV, m, 1), jnp.float32),
            pltpu.SemaphoreType.DMA((N_DEV - 1,)),
            pltpu.SemaphoreType.DMA((N_DEV - 1,)),
        ],
    )(x, gamma_row)


# device time: 14439 ns/iter; 3.6436x vs baseline; 3.6436x over previous
import jax
import jax.numpy as jnp
from jax import lax
from jax.experimental import pallas as pl
from jax.experimental.pallas import tpu as pltpu

N_DEV = 8
EPS = 1e-5
N_GLOBAL = 4096


def kernel(x, gamma):
    m, n = x.shape
    gamma_row = gamma.reshape(1, n)

    def body(x_ref, g_ref, out_ref, comm_ref, send_sems, recv_sems):
        my_pos = lax.axis_index("i")

        xv = x_ref[:, :]
        partial = jnp.sum(xv * xv, axis=1).reshape(1, m)
        comm_ref[pl.ds(my_pos, 1), :, :] = partial[None]

        for k in range(1, N_DEV):
            tgt = lax.rem(my_pos + k, N_DEV)
            rdma = pltpu.make_async_remote_copy(
                src_ref=comm_ref.at[my_pos],
                dst_ref=comm_ref.at[my_pos],
                send_sem=send_sems.at[k - 1],
                recv_sem=recv_sems.at[my_pos],
                device_id=(tgt,),
                device_id_type=pl.DeviceIdType.MESH,
            )
            rdma.start()

        for k in range(1, N_DEV):
            src = lax.rem(my_pos + k, N_DEV)
            recv = pltpu.make_async_remote_copy(
                src_ref=comm_ref.at[src],
                dst_ref=comm_ref.at[src],
                send_sem=send_sems.at[k - 1],
                recv_sem=recv_sems.at[src],
                device_id=(src,),
                device_id_type=pl.DeviceIdType.MESH,
            )
            recv.wait_recv()

        total = jnp.sum(comm_ref[:, 0, :], axis=0)
        inv = lax.rsqrt(total / N_GLOBAL + EPS).reshape(m, 1)
        out_ref[:, :] = xv * g_ref[0, :][None, :] * inv

        for k in range(1, N_DEV):
            send = pltpu.make_async_remote_copy(
                src_ref=comm_ref.at[my_pos],
                dst_ref=comm_ref.at[my_pos],
                send_sem=send_sems.at[k - 1],
                recv_sem=recv_sems.at[my_pos],
                device_id=(lax.rem(my_pos + k, N_DEV),),
                device_id_type=pl.DeviceIdType.MESH,
            )
            send.wait_send()

    return pl.pallas_call(
        body,
        out_shape=jax.ShapeDtypeStruct((m, n), jnp.float32),
        in_specs=[
            pl.BlockSpec(memory_space=pltpu.VMEM),
            pl.BlockSpec(memory_space=pltpu.VMEM),
        ],
        out_specs=pl.BlockSpec(memory_space=pltpu.VMEM),
        scratch_shapes=[
            pltpu.VMEM((N_DEV, 1, m), jnp.float32),
            pltpu.SemaphoreType.DMA((N_DEV - 1,)),
            pltpu.SemaphoreType.DMA((N_DEV,)),
        ],
    )(x, gamma_row)


# device time: 13704 ns/iter; 3.8390x vs baseline; 1.0536x over previous
import jax
import jax.numpy as jnp
from jax import lax
from jax.experimental import pallas as pl
from jax.experimental.pallas import tpu as pltpu

N_DEV = 8
EPS = 1e-5
N_GLOBAL = 4096
MASKS = (1, 3, 4)


def kernel(x, gamma):
    m, n = x.shape
    gamma_row = gamma.reshape(1, n)

    def body(x_ref, g_ref, out_ref, sbuf, rbuf, send_sems, recv_sems):
        my_pos = lax.axis_index("i")

        barrier_sem = pltpu.get_barrier_semaphore()
        for mask in MASKS:
            pl.semaphore_signal(
                barrier_sem, inc=1,
                device_id=(my_pos ^ mask,),
                device_id_type=pl.DeviceIdType.MESH,
            )
        pl.semaphore_wait(barrier_sem, len(MASKS))

        xv = x_ref[:, :]
        acc = jnp.sum(xv * xv, axis=1).reshape(1, m)

        xg = None
        for r, mask in enumerate(MASKS):
            partner = my_pos ^ mask
            sbuf[r, :, :] = acc
            rdma = pltpu.make_async_remote_copy(
                src_ref=sbuf.at[r],
                dst_ref=rbuf.at[r],
                send_sem=send_sems.at[r],
                recv_sem=recv_sems.at[r],
                device_id=(partner,),
                device_id_type=pl.DeviceIdType.MESH,
            )
            rdma.start()
            if r == 0:
                xg = xv * g_ref[0, :][None, :]
            rdma.wait()
            acc = acc + rbuf[r, :, :]

        inv = lax.rsqrt(acc / N_GLOBAL + EPS).reshape(m, 1)
        out_ref[:, :] = xg * inv

    return pl.pallas_call(
        body,
        out_shape=jax.ShapeDtypeStruct((m, n), jnp.float32),
        in_specs=[
            pl.BlockSpec(memory_space=pltpu.VMEM),
            pl.BlockSpec(memory_space=pltpu.VMEM),
        ],
        out_specs=pl.BlockSpec(memory_space=pltpu.VMEM),
        scratch_shapes=[
            pltpu.VMEM((3, 1, m), jnp.float32),
            pltpu.VMEM((3, 1, m), jnp.float32),
            pltpu.SemaphoreType.DMA((3,)),
            pltpu.SemaphoreType.DMA((3,)),
        ],
        compiler_params=pltpu.CompilerParams(collective_id=0),
    )(x, gamma_row)
